# baseline (speedup 1.0000x reference)
import functools

import jax
import jax.numpy as jnp
from jax.experimental import pallas as pl


def _dense_moe_body(ids_ref, w_ref, x_ref, g_ref, u_ref, d_ref, out_ref,
                    *, n_m_tiles):
    e = pl.program_id(0)
    m = pl.program_id(1)

    @pl.when(jnp.logical_and(e == 0, m == 0))
    def _init():
        out_ref[...] = jnp.zeros_like(out_ref)

    x = x_ref[...]
    gw = g_ref[0]
    uw = u_ref[0]
    dw = d_ref[0]

    g = jax.lax.dot_general(x, gw, (((1,), (1,)), ((), ())),
                            preferred_element_type=jnp.float32)
    u = jax.lax.dot_general(x, uw, (((1,), (1,)), ((), ())),
                            preferred_element_type=jnp.float32)
    h = (g * jax.lax.logistic(g) * u).astype(jnp.bfloat16)
    y = jax.lax.dot_general(h, dw, (((1,), (0,)), ((), ())),
                            preferred_element_type=jnp.float32)

    ids = ids_ref[...]
    w = w_ref[...]
    c = jnp.sum(jnp.where(ids == e, w, 0.0), axis=1)
    out_ref[...] += c[:, None] * y


def kernel(input_tensor, expert_ids, weights, gate_w, up_w, down_w):
    B, H = input_tensor.shape
    E, M, _ = gate_w.shape
    MT = 352
    n_m_tiles = M // MT

    x16 = input_tensor.astype(jnp.bfloat16)
    g16 = gate_w.astype(jnp.bfloat16)
    u16 = up_w.astype(jnp.bfloat16)
    d16 = down_w.astype(jnp.bfloat16).transpose(0, 2, 1)
    ids = expert_ids.astype(jnp.int32)

    grid = (E, n_m_tiles)
    out = pl.pallas_call(
        functools.partial(_dense_moe_body, n_m_tiles=n_m_tiles),
        grid=grid,
        in_specs=[
            pl.BlockSpec((B, ids.shape[1]), lambda e, m: (0, 0)),
            pl.BlockSpec((B, weights.shape[1]), lambda e, m: (0, 0)),
            pl.BlockSpec((B, H), lambda e, m: (0, 0)),
            pl.BlockSpec((1, MT, H), lambda e, m: (e, m, 0)),
            pl.BlockSpec((1, MT, H), lambda e, m: (e, m, 0)),
            pl.BlockSpec((1, MT, H), lambda e, m: (e, m, 0)),
        ],
        out_specs=pl.BlockSpec((B, H), lambda e, m: (0, 0)),
        out_shape=jax.ShapeDtypeStruct((B, H), jnp.float32),
    )(ids, weights, x16, g16, u16, d16)
    return out
